# baseline (speedup 1.0000x reference)
import functools

import jax
import jax.numpy as jnp
from jax import lax
from jax.experimental import pallas as pl
from jax.experimental.pallas import tpu as pltpu

N, D, H, L, K, T = 8192, 1024, 512, 64, 16, 50
DF = 1.0
C = 512
NC = N // C
NBUF = 4


def _in_copy(x_hbm, x_v, in_sem, c, slot):
    return pltpu.make_async_copy(
        x_hbm.at[pl.ds(c * C, C), :], x_v.at[slot], in_sem.at[slot])


def _out_copies(refs, bufs, sems, c, slot):
    (z_hbm, q_hbm, s_hbm, xh_hbm, r_hbm) = refs
    (z_v, q_v, s_v, xh_v, r_v) = bufs
    (z_sem, q_sem, s_sem, xh_sem, r_sem) = sems
    rows = pl.ds(c * C, C)
    return (
        pltpu.make_async_copy(xh_v.at[slot], xh_hbm.at[rows, :], xh_sem.at[slot]),
        pltpu.make_async_copy(z_v.at[slot], z_hbm.at[rows, :], z_sem.at[slot]),
        pltpu.make_async_copy(q_v.at[slot], q_hbm.at[rows, :], q_sem.at[slot]),
        pltpu.make_async_copy(s_v.at[slot], s_hbm.at[rows, :], s_sem.at[slot]),
        pltpu.make_async_copy(r_v.at[slot], r_hbm.at[rows, :], r_sem.at[slot]),
    )


def _body(x_hbm, w1_ref, b1_ref, w2_ref, b2_ref, decw_ref, decb_ref,
          swz_ref, swx_ref, sb_ref, c_ref,
          z_hbm, q_hbm, s_hbm, xh_hbm, r_hbm,
          x_v, xh_v, z_v, q_v, s_v, r_v,
          in_sem, xh_sem, z_sem, q_sem, s_sem, r_sem):
    out_refs = (z_hbm, q_hbm, s_hbm, xh_hbm, r_hbm)
    out_bufs = (z_v, q_v, s_v, xh_v, r_v)
    out_sems = (z_sem, q_sem, s_sem, xh_sem, r_sem)

    w1 = w1_ref[...].astype(jnp.bfloat16)
    w2 = w2_ref[...].astype(jnp.bfloat16)
    decw = decw_ref[...].astype(jnp.bfloat16)
    swz = swz_ref[...].astype(jnp.bfloat16)
    swx = swx_ref[...].astype(jnp.bfloat16)
    cc = c_ref[...]
    c2 = jnp.sum(cc * cc, axis=1)[None, :]

    for s in range(NBUF):
        _in_copy(x_hbm, x_v, in_sem, s, s).start()

    def step(c, carry):
        slot = lax.rem(c, NBUF)
        _in_copy(x_hbm, x_v, in_sem, c, slot).wait()

        x = x_v[slot]
        xb = x.astype(jnp.bfloat16)
        h1 = jnp.maximum(
            jnp.dot(xb, w1, preferred_element_type=jnp.float32)
            + b1_ref[...], 0.0)
        z = jnp.dot(h1.astype(jnp.bfloat16), w2,
                    preferred_element_type=jnp.float32) + b2_ref[...]
        zb = z.astype(jnp.bfloat16)

        zc = lax.dot_general(z, cc, (((1,), (1,)), ((), ())),
                             preferred_element_type=jnp.float32)
        z2 = jnp.sum(z * z, axis=1, keepdims=True)
        dist2 = jnp.maximum(z2 - 2.0 * zc + c2, 0.0)
        logits = -0.5 * (DF + 1.0) * jnp.log1p(dist2 / DF)
        logits = logits - jnp.max(logits, axis=1, keepdims=True)
        e = jnp.exp(logits)
        q = e / jnp.sum(e, axis=1, keepdims=True)

        surv = (jnp.dot(zb, swz, preferred_element_type=jnp.float32)
                + jnp.dot(xb, swx, preferred_element_type=jnp.float32)
                + sb_ref[...])

        x_hat = (jnp.dot(zb, decw, preferred_element_type=jnp.float32)
                 + decb_ref[...])
        d = x_hat - x
        rec = jnp.sum(d * d, axis=1, keepdims=True) * (1.0 / D)

        @pl.when(c >= NBUF)
        def _():
            for cp in _out_copies(out_refs, out_bufs, out_sems, c - NBUF, slot):
                cp.wait()

        z_v[slot] = z
        q_v[slot] = q
        s_v[slot] = surv
        xh_v[slot] = x_hat
        r_v[slot] = rec
        for cp in _out_copies(out_refs, out_bufs, out_sems, c, slot):
            cp.start()

        @pl.when(c + NBUF < NC)
        def _():
            _in_copy(x_hbm, x_v, in_sem, c + NBUF, slot).start()
        return carry

    lax.fori_loop(0, NC, step, 0)

    for c in range(NC - NBUF, NC):
        for cp in _out_copies(out_refs, out_bufs, out_sems, c, c % NBUF):
            cp.wait()


@jax.jit
def kernel(x, enc_W1, enc_b1, enc_W2, enc_b2, dec_W, dec_b, surv_W, surv_b, centers):
    anyspec = pl.BlockSpec(memory_space=pl.ANY)
    vfull = pl.BlockSpec(memory_space=pltpu.MemorySpace.VMEM)

    z, q, surv, x_hat, rec = pl.pallas_call(
        _body,
        in_specs=[anyspec,
                  vfull, vfull, vfull, vfull, vfull, vfull,
                  vfull, vfull, vfull, vfull],
        out_specs=[anyspec] * 5,
        out_shape=[
            jax.ShapeDtypeStruct((N, L), jnp.float32),
            jax.ShapeDtypeStruct((N, K), jnp.float32),
            jax.ShapeDtypeStruct((N, T), jnp.float32),
            jax.ShapeDtypeStruct((N, D), jnp.float32),
            jax.ShapeDtypeStruct((N, 1), jnp.float32),
        ],
        scratch_shapes=[
            pltpu.VMEM((NBUF, C, D), jnp.float32),
            pltpu.VMEM((NBUF, C, D), jnp.float32),
            pltpu.VMEM((NBUF, C, L), jnp.float32),
            pltpu.VMEM((NBUF, C, K), jnp.float32),
            pltpu.VMEM((NBUF, C, T), jnp.float32),
            pltpu.VMEM((NBUF, C, 1), jnp.float32),
            pltpu.SemaphoreType.DMA((NBUF,)),
            pltpu.SemaphoreType.DMA((NBUF,)),
            pltpu.SemaphoreType.DMA((NBUF,)),
            pltpu.SemaphoreType.DMA((NBUF,)),
            pltpu.SemaphoreType.DMA((NBUF,)),
            pltpu.SemaphoreType.DMA((NBUF,)),
        ],
    )(x, enc_W1, enc_b1[None, :], enc_W2, enc_b2[None, :],
      dec_W, dec_b[None, :], surv_W[:L], surv_W[L:], surv_b[None, :],
      centers)

    zeros_nl = jnp.zeros((N, L), jnp.float32)
    kld = jnp.zeros((N,), jnp.float32)
    return (z, zeros_nl, zeros_nl, kld, x_hat, rec[:, 0], q, surv, centers)


# speedup vs baseline: 1.0248x; 1.0248x over previous
import functools

import jax
import jax.numpy as jnp
from jax import lax
from jax.experimental import pallas as pl
from jax.experimental.pallas import tpu as pltpu

N, D, H, L, K, T = 8192, 1024, 512, 64, 16, 50
DF = 1.0
C = 1024
NC = N // C
NBUF = 3


def _in_copy(x_hbm, x_v, in_sem, c, slot):
    return pltpu.make_async_copy(
        x_hbm.at[pl.ds(c * C, C), :], x_v.at[slot], in_sem.at[slot])


def _out_copies(refs, bufs, sems, c, slot):
    (z_hbm, q_hbm, s_hbm, xh_hbm, r_hbm) = refs
    (z_v, q_v, s_v, xh_v, r_v) = bufs
    (z_sem, q_sem, s_sem, xh_sem, r_sem) = sems
    rows = pl.ds(c * C, C)
    return (
        pltpu.make_async_copy(xh_v.at[slot], xh_hbm.at[rows, :], xh_sem.at[slot]),
        pltpu.make_async_copy(z_v.at[slot], z_hbm.at[rows, :], z_sem.at[slot]),
        pltpu.make_async_copy(q_v.at[slot], q_hbm.at[rows, :], q_sem.at[slot]),
        pltpu.make_async_copy(s_v.at[slot], s_hbm.at[rows, :], s_sem.at[slot]),
        pltpu.make_async_copy(r_v.at[slot], r_hbm.at[rows, :], r_sem.at[slot]),
    )


def _body(x_hbm, w1_ref, b1_ref, w2_ref, b2_ref, decw_ref, decb_ref,
          swz_ref, swx_ref, sb_ref, c_ref,
          z_hbm, q_hbm, s_hbm, xh_hbm, r_hbm,
          x_v, xh_v, z_v, q_v, s_v, r_v,
          in_sem, xh_sem, z_sem, q_sem, s_sem, r_sem):
    out_refs = (z_hbm, q_hbm, s_hbm, xh_hbm, r_hbm)
    out_bufs = (z_v, q_v, s_v, xh_v, r_v)
    out_sems = (z_sem, q_sem, s_sem, xh_sem, r_sem)

    w1 = w1_ref[...].astype(jnp.bfloat16)
    w2 = w2_ref[...].astype(jnp.bfloat16)
    decw = decw_ref[...].astype(jnp.bfloat16)
    swz = swz_ref[...].astype(jnp.bfloat16)
    swx = swx_ref[...].astype(jnp.bfloat16)
    cc = c_ref[...]
    c2 = jnp.sum(cc * cc, axis=1)[None, :]

    for s in range(NBUF):
        _in_copy(x_hbm, x_v, in_sem, s, s).start()

    def step(c, carry):
        slot = lax.rem(c, NBUF)
        _in_copy(x_hbm, x_v, in_sem, c, slot).wait()

        x = x_v[slot]
        xb = x.astype(jnp.bfloat16)
        h1 = jnp.maximum(
            jnp.dot(xb, w1, preferred_element_type=jnp.float32)
            + b1_ref[...], 0.0)
        z = jnp.dot(h1.astype(jnp.bfloat16), w2,
                    preferred_element_type=jnp.float32) + b2_ref[...]
        zb = z.astype(jnp.bfloat16)

        zc = lax.dot_general(z, cc, (((1,), (1,)), ((), ())),
                             preferred_element_type=jnp.float32)
        z2 = jnp.sum(z * z, axis=1, keepdims=True)
        dist2 = jnp.maximum(z2 - 2.0 * zc + c2, 0.0)
        logits = -0.5 * (DF + 1.0) * jnp.log1p(dist2 / DF)
        logits = logits - jnp.max(logits, axis=1, keepdims=True)
        e = jnp.exp(logits)
        q = e / jnp.sum(e, axis=1, keepdims=True)

        surv = (jnp.dot(zb, swz, preferred_element_type=jnp.float32)
                + jnp.dot(xb, swx, preferred_element_type=jnp.float32)
                + sb_ref[...])

        x_hat = (jnp.dot(zb, decw, preferred_element_type=jnp.float32)
                 + decb_ref[...])
        d = x_hat - x
        rec = jnp.sum(d * d, axis=1, keepdims=True) * (1.0 / D)

        @pl.when(c >= NBUF)
        def _():
            for cp in _out_copies(out_refs, out_bufs, out_sems, c - NBUF, slot):
                cp.wait()

        z_v[slot] = z
        q_v[slot] = q
        s_v[slot] = surv
        xh_v[slot] = x_hat
        r_v[slot] = rec
        for cp in _out_copies(out_refs, out_bufs, out_sems, c, slot):
            cp.start()

        @pl.when(c + NBUF < NC)
        def _():
            _in_copy(x_hbm, x_v, in_sem, c + NBUF, slot).start()
        return carry

    lax.fori_loop(0, NC, step, 0)

    for c in range(NC - NBUF, NC):
        for cp in _out_copies(out_refs, out_bufs, out_sems, c, c % NBUF):
            cp.wait()


@jax.jit
def kernel(x, enc_W1, enc_b1, enc_W2, enc_b2, dec_W, dec_b, surv_W, surv_b, centers):
    anyspec = pl.BlockSpec(memory_space=pl.ANY)
    vfull = pl.BlockSpec(memory_space=pltpu.MemorySpace.VMEM)

    z, q, surv, x_hat, rec = pl.pallas_call(
        _body,
        in_specs=[anyspec,
                  vfull, vfull, vfull, vfull, vfull, vfull,
                  vfull, vfull, vfull, vfull],
        out_specs=[anyspec] * 5,
        out_shape=[
            jax.ShapeDtypeStruct((N, L), jnp.float32),
            jax.ShapeDtypeStruct((N, K), jnp.float32),
            jax.ShapeDtypeStruct((N, T), jnp.float32),
            jax.ShapeDtypeStruct((N, D), jnp.float32),
            jax.ShapeDtypeStruct((N, 1), jnp.float32),
        ],
        scratch_shapes=[
            pltpu.VMEM((NBUF, C, D), jnp.float32),
            pltpu.VMEM((NBUF, C, D), jnp.float32),
            pltpu.VMEM((NBUF, C, L), jnp.float32),
            pltpu.VMEM((NBUF, C, K), jnp.float32),
            pltpu.VMEM((NBUF, C, T), jnp.float32),
            pltpu.VMEM((NBUF, C, 1), jnp.float32),
            pltpu.SemaphoreType.DMA((NBUF,)),
            pltpu.SemaphoreType.DMA((NBUF,)),
            pltpu.SemaphoreType.DMA((NBUF,)),
            pltpu.SemaphoreType.DMA((NBUF,)),
            pltpu.SemaphoreType.DMA((NBUF,)),
            pltpu.SemaphoreType.DMA((NBUF,)),
        ],
    )(x, enc_W1, enc_b1[None, :], enc_W2, enc_b2[None, :],
      dec_W, dec_b[None, :], surv_W[:L], surv_W[L:], surv_b[None, :],
      centers)

    zeros_nl = jnp.zeros((N, L), jnp.float32)
    kld = jnp.zeros((N,), jnp.float32)
    return (z, zeros_nl, zeros_nl, kld, x_hat, rec[:, 0], q, surv, centers)
